# baseline (speedup 1.0000x reference)
import jax
import jax.numpy as jnp
from jax import lax
from jax.experimental import pallas as pl
from jax.experimental.pallas import tpu as pltpu
from jax.experimental.pallas import tpu_sc as plsc

NUM_EMB = 991
VOCAB = 1024
EMB = 64
BATCH = 4096
LIST = 200
NGRP = 13

NCORES = 2
NSUB = 16
NW = NCORES * NSUB
NSPLIT = 2
SPLIT = BATCH // NSPLIT
BPW = SPLIT // NW
CHUNK = 32
NCHUNK = BPW // CHUNK

MM_BLK = 512


def _sc_hist_body(
    idx_hbm, zeros_hbm, hist_hbm, idx_v, h0, h1, sem0, sem1, zsem0, zsem1
):
    cid = lax.axis_index("c")
    sid = lax.axis_index("s")
    wid = sid * NCORES + cid
    base = wid * BPW

    zcopies = [
        pltpu.async_copy(zeros_hbm, h0, zsem0),
        pltpu.async_copy(zeros_hbm, h1, zsem1),
    ]

    pltpu.sync_copy(idx_hbm.at[pl.ds(base, BPW)], idx_v)

    ones = jnp.ones((16,), jnp.float32)
    lanes = lax.iota(jnp.int32, 16)
    tail_mask = lanes >= 8

    bufs = (h0, h1)
    sems = (sem0, sem1)

    def do_chunk(c, buf, sem):
        def row_hist(r, _):
            row = c * CHUNK + r
            hist_off = r * VOCAB
            for g in range(NGRP):
                if g < NGRP - 1:
                    sidx = idx_v[row, pl.ds(g * 16, 16)]
                    plsc.addupdate_scatter(buf, [sidx + hist_off], ones)
                else:
                    sidx = idx_v[row, pl.ds(LIST - 16, 16)]
                    plsc.addupdate_scatter(
                        buf, [sidx + hist_off], ones, mask=tail_mask
                    )
            return 0

        lax.fori_loop(0, CHUNK, row_hist, 0)

        return pltpu.async_copy(
            buf,
            hist_hbm.at[pl.ds((base + c * CHUNK) * VOCAB, CHUNK * VOCAB)],
            sem,
        )

    assert NCHUNK == 2
    copies = []
    for c in range(NCHUNK):
        zcopies[c].wait()
        copies.append(do_chunk(c, bufs[c], sems[c]))
    copies[0].wait()
    copies[1].wait()


def _mm_body(h_ref, t_ref, o_ref):
    o_ref[...] = jnp.dot(
        h_ref[...].reshape(MM_BLK, VOCAB),
        t_ref[...],
        preferred_element_type=jnp.float32,
    ) * jnp.float32(1.0 / LIST)


@jax.jit
def kernel(wyck_x, embedding_table):
    idx = wyck_x[:, -1]
    tpad = jnp.zeros((VOCAB, EMB), jnp.float32).at[:NUM_EMB].set(
        embedding_table
    )

    mesh = plsc.VectorSubcoreMesh(core_axis_name="c", subcore_axis_name="s")
    hist_call = pl.kernel(
        _sc_hist_body,
        out_type=jax.ShapeDtypeStruct((SPLIT * VOCAB,), jnp.float32),
        mesh=mesh,
        compiler_params=pltpu.CompilerParams(needs_layout_passes=False),
        scratch_types=[
            pltpu.VMEM((BPW, LIST), jnp.int32),
            pltpu.VMEM((CHUNK * VOCAB,), jnp.float32),
            pltpu.VMEM((CHUNK * VOCAB,), jnp.float32),
            pltpu.SemaphoreType.DMA,
            pltpu.SemaphoreType.DMA,
            pltpu.SemaphoreType.DMA,
            pltpu.SemaphoreType.DMA,
        ],
    )
    zblk = jnp.zeros((CHUNK * VOCAB,), jnp.float32)

    def mm_call(h):
        return pl.pallas_call(
            _mm_body,
            out_shape=jax.ShapeDtypeStruct((SPLIT, EMB), jnp.float32),
            grid=(SPLIT // MM_BLK,),
            in_specs=[
                pl.BlockSpec((MM_BLK * VOCAB,), lambda i: (i,)),
                pl.BlockSpec((VOCAB, EMB), lambda i: (0, 0)),
            ],
            out_specs=pl.BlockSpec((MM_BLK, EMB), lambda i: (i, 0)),
        )(h, tpad)

    hists = [
        hist_call(idx[s * SPLIT:(s + 1) * SPLIT], zblk)
        for s in range(NSPLIT)
    ]
    outs = [mm_call(h) for h in hists]
    return jnp.concatenate(outs, axis=0)


# speedup vs baseline: 1.4727x; 1.4727x over previous
import jax
import jax.numpy as jnp
from jax import lax
from jax.experimental import pallas as pl
from jax.experimental.pallas import tpu as pltpu
from jax.experimental.pallas import tpu_sc as plsc

NUM_EMB = 991
VOCAB = 1024
PACK = 4
PBINS = VOCAB // PACK
EMB = 64
BATCH = 4096
LIST = 200
NGRP = 13
NCORES = 2
NSUB = 16
NW = NCORES * NSUB
NSPLIT = 2
SPLIT = BATCH // NSPLIT
BPW = SPLIT // NW
CHUNK = 32
NCHUNK = BPW // CHUNK

MM_BLK = 512


def _sc_hist_body(idx_hbm, hist_hbm, idx_v, h0, h1, sem0, sem1):
    cid = lax.axis_index("c")
    sid = lax.axis_index("s")
    wid = sid * NCORES + cid
    base = wid * BPW

    pltpu.sync_copy(idx_hbm.at[pl.ds(base, BPW)], idx_v)

    zeros = jnp.zeros((16,), jnp.int32)
    one = jnp.full((16,), 1, jnp.int32)
    three = jnp.full((16,), 3, jnp.int32)
    eight = jnp.full((16,), 8, jnp.int32)
    lanes = lax.iota(jnp.int32, 16)
    tail_mask = lanes >= 8

    bufs = (h0, h1)
    sems = (sem0, sem1)

    def do_chunk(c, buf, sem):
        def zero_one(z, _):
            for u in range(16):
                buf[pl.ds(z * 256 + u * 16, 16)] = zeros
            return 0

        lax.fori_loop(0, CHUNK * PBINS // 256, zero_one, 0)

        def row_hist(r, _):
            row = c * CHUNK + r
            hist_off = r * PBINS
            for g in range(NGRP):
                if g < NGRP - 1:
                    sidx = idx_v[row, pl.ds(g * 16, 16)]
                    mask = None
                else:
                    sidx = idx_v[row, pl.ds(LIST - 16, 16)]
                    mask = tail_mask
                pbin = lax.shift_right_logical(sidx, 2)
                addend = lax.shift_left(
                    one, lax.shift_left(sidx & three, 3)
                )
                if mask is None:
                    plsc.addupdate_scatter(
                        buf, [pbin + hist_off], addend
                    )
                else:
                    plsc.addupdate_scatter(
                        buf, [pbin + hist_off], addend, mask=mask
                    )
            return 0

        lax.fori_loop(0, CHUNK, row_hist, 0)

        return pltpu.async_copy(
            buf,
            hist_hbm.at[pl.ds((base + c * CHUNK) * PBINS, CHUNK * PBINS)],
            sem,
        )

    copies = []
    for c in range(NCHUNK):
        if c >= 2:
            copies[c - 2].wait()
        copies.append(do_chunk(c, bufs[c % 2], sems[c % 2]))
    copies[-2].wait()
    copies[-1].wait()


def _mm_body(h_ref, t_ref, o_ref):
    h = h_ref[...].reshape(MM_BLK, PBINS)
    acc = jnp.zeros((MM_BLK, EMB), jnp.float32)
    for k in range(PACK):
        ck = lax.shift_right_logical(h, 8 * k)
        if k < PACK - 1:
            ck = ck & 0xFF
        acc += jnp.dot(
            ck.astype(jnp.float32),
            t_ref[k],
            preferred_element_type=jnp.float32,
        )
    o_ref[...] = acc * jnp.float32(1.0 / LIST)


@jax.jit
def kernel(wyck_x, embedding_table):
    idx = wyck_x[:, -1]
    tpad = jnp.zeros((VOCAB, EMB), jnp.float32).at[:NUM_EMB].set(
        embedding_table
    )
    tgrp = tpad.reshape(PBINS, PACK, EMB).transpose(1, 0, 2)

    mesh = plsc.VectorSubcoreMesh(core_axis_name="c", subcore_axis_name="s")
    hist_call = pl.kernel(
        _sc_hist_body,
        out_type=jax.ShapeDtypeStruct((SPLIT * PBINS,), jnp.int32),
        mesh=mesh,
        compiler_params=pltpu.CompilerParams(needs_layout_passes=False),
        scratch_types=[
            pltpu.VMEM((BPW, LIST), jnp.int32),
            pltpu.VMEM((CHUNK * PBINS,), jnp.int32),
            pltpu.VMEM((CHUNK * PBINS,), jnp.int32),
            pltpu.SemaphoreType.DMA,
            pltpu.SemaphoreType.DMA,
        ],
    )

    def mm_call(h):
        return pl.pallas_call(
            _mm_body,
            out_shape=jax.ShapeDtypeStruct((SPLIT, EMB), jnp.float32),
            grid=(SPLIT // MM_BLK,),
            in_specs=[
                pl.BlockSpec((MM_BLK * PBINS,), lambda i: (i,)),
                pl.BlockSpec((PACK, PBINS, EMB), lambda i: (0, 0, 0)),
            ],
            out_specs=pl.BlockSpec((MM_BLK, EMB), lambda i: (i, 0)),
        )(h, tgrp)

    hists = [
        hist_call(idx[s * SPLIT:(s + 1) * SPLIT]) for s in range(NSPLIT)
    ]
    outs = [mm_call(h) for h in hists]
    return jnp.concatenate(outs, axis=0)
